# baseline (speedup 1.0000x reference)
import functools

import jax
import jax.numpy as jnp
from jax import lax
from jax.experimental import pallas as pl
from jax.experimental.pallas import tpu as pltpu
from jax.experimental.pallas import tpu_sc as plsc

B = 2
S = 2048
T = B * S
D = 1024
F = 2048
E = 8
K = 2
A = T * K
EP = 128

TM = 256
P = A + E * TM
NT = P // TM
FCH = 512
NFC = F // FCH

NC = 2
NS = 16
NW = NC * NS


def _linear_body(x_ref, wl_ref, b_ref, wg_ref, flat_ref, log_ref, sent_ref,
                 acc_ref):
    i = pl.program_id(0)

    @pl.when(i == 0)
    def _():
        acc_ref[...] = jnp.zeros_like(acc_ref)

    acc = jnp.dot(x_ref[...], wl_ref[...],
                  preferred_element_type=jnp.float32) + b_ref[...]
    flat_ref[...] = acc
    log_ref[...] = jnp.dot(acc, wg_ref[...],
                           preferred_element_type=jnp.float32)

    part = jnp.sum(acc, axis=0, keepdims=True)
    b = i // (S // TM)
    rowi = lax.broadcasted_iota(jnp.int32, (8, D), 0)
    acc_ref[...] += jnp.where(rowi == b, part, 0.0)

    @pl.when(i == T // TM - 1)
    def _():
        sent_ref[...] = acc_ref[...]


def _linear(x2, W_lin, b_lin, Wg1):
    return pl.pallas_call(
        _linear_body,
        grid=(T // TM,),
        in_specs=[
            pl.BlockSpec((TM, D), lambda i: (i, 0)),
            pl.BlockSpec((D, D), lambda i: (0, 0)),
            pl.BlockSpec((1, D), lambda i: (0, 0)),
            pl.BlockSpec((D, E), lambda i: (0, 0)),
        ],
        out_specs=[
            pl.BlockSpec((TM, D), lambda i: (i, 0)),
            pl.BlockSpec((TM, E), lambda i: (i, 0)),
            pl.BlockSpec((8, D), lambda i: (0, 0)),
        ],
        out_shape=[
            jax.ShapeDtypeStruct((T, D), jnp.float32),
            jax.ShapeDtypeStruct((T, E), jnp.float32),
            jax.ShapeDtypeStruct((8, D), jnp.float32),
        ],
        scratch_shapes=[pltpu.VMEM((8, D), jnp.float32)],
    )(x2, W_lin, b_lin.reshape(1, D), Wg1)


def _route(logits):
    probs = jax.nn.softmax(logits, axis=-1)
    topv, topi = lax.top_k(probs, K)
    gates = topv / jnp.sum(topv, axis=-1, keepdims=True)

    e = topi.reshape(-1).astype(jnp.int32)
    g = gates.reshape(-1)
    oh = (e[:, None] == jnp.arange(E, dtype=jnp.int32)).astype(jnp.int32)
    cum = jnp.cumsum(oh, axis=0)
    rank = cum[jnp.arange(A), e] - 1
    counts = cum[-1]
    padded = ((counts + TM - 1) // TM) * TM
    ends = jnp.cumsum(padded)
    gstart = ends - padded
    dest = (gstart[e] + rank).astype(jnp.int32)

    tok = jnp.arange(A, dtype=jnp.int32) // K
    row_index = jnp.zeros((P,), jnp.int32).at[dest].set(tok)
    gate_s = jnp.zeros((P,), jnp.float32).at[dest].set(g)
    tile_expert = jnp.searchsorted(
        ends, jnp.arange(NT, dtype=jnp.int32) * TM, side='right')
    tile_expert = jnp.minimum(tile_expert, E - 1).astype(jnp.int32)
    inv = dest.reshape(T, K)
    return row_index, gate_s.reshape(P, 1), tile_expert, inv[:, 0], inv[:, 1]


_GCH = 32
_GROWS = P // NW
_GNCH = _GROWS // _GCH


@functools.cache
def _build_sc_gather():
    mesh = plsc.VectorSubcoreMesh(core_axis_name="c", subcore_axis_name="s")

    @functools.partial(
        pl.kernel,
        mesh=mesh,
        out_type=jax.ShapeDtypeStruct((P, D), jnp.float32),
        scratch_types=[
            pltpu.VMEM((_GROWS,), jnp.int32),
            [pltpu.VMEM((_GCH, D), jnp.float32) for _ in range(3)],
            pltpu.SemaphoreType.DMA,
            [pltpu.SemaphoreType.DMA for _ in range(3)],
            [pltpu.SemaphoreType.DMA for _ in range(3)],
        ],
    )
    def gather_k(src_hbm, idx_hbm, out_hbm, idx_v, bufs, isem, gsems, ssems):
        wid = lax.axis_index("s") * NC + lax.axis_index("c")
        base = wid * _GROWS
        pltpu.async_copy(idx_hbm.at[pl.ds(base, _GROWS)], idx_v, isem).wait()

        ghandle = {}
        shandle = {}

        def start_gather(c):
            sl = idx_v.at[pl.ds(c * _GCH, _GCH)]
            ghandle[c] = pltpu.async_copy(
                src_hbm.at[sl], bufs[c % 3], gsems[c % 3])

        start_gather(0)
        for c in range(_GNCH):
            if c + 1 < _GNCH:
                if c + 1 >= 3:
                    shandle[c + 1 - 3].wait()
                start_gather(c + 1)
            ghandle[c].wait()
            shandle[c] = pltpu.async_copy(
                bufs[c % 3], out_hbm.at[pl.ds(base + c * _GCH, _GCH)],
                ssems[c % 3])
        for c in range(_GNCH - 3, _GNCH):
            shandle[c].wait()

    return gather_k


def _sc_gather(src, idx):
    return _build_sc_gather()(src, idx)


_CCH = 16
_CNCH = _GROWS // _CCH


@functools.cache
def _build_sc_gather_combine():
    mesh = plsc.VectorSubcoreMesh(core_axis_name="c", subcore_axis_name="s")

    @functools.partial(
        pl.kernel,
        mesh=mesh,
        out_type=jax.ShapeDtypeStruct((P, D), jnp.float32),
        scratch_types=[
            pltpu.VMEM((_GROWS,), jnp.int32),
            pltpu.VMEM((_GROWS,), jnp.int32),
            [pltpu.VMEM((_CCH, D), jnp.float32) for _ in range(2)],
            [pltpu.VMEM((_CCH, D), jnp.float32) for _ in range(2)],
            pltpu.SemaphoreType.DMA,
            [pltpu.SemaphoreType.DMA for _ in range(2)],
            [pltpu.SemaphoreType.DMA for _ in range(2)],
        ],
    )
    def gc_k(yg_hbm, j0_hbm, j1_hbm, out_hbm, j0_v, j1_v, abufs, bbufs,
             isem, gsems, ssems):
        wid = lax.axis_index("s") * NC + lax.axis_index("c")
        base = wid * _GROWS
        pltpu.async_copy(j0_hbm.at[pl.ds(base, _GROWS)], j0_v, isem).wait()
        pltpu.async_copy(j1_hbm.at[pl.ds(base, _GROWS)], j1_v, isem).wait()

        ghandle = {}
        shandle = {}

        def start_gathers(c):
            p = c % 2
            sl0 = j0_v.at[pl.ds(c * _CCH, _CCH)]
            sl1 = j1_v.at[pl.ds(c * _CCH, _CCH)]
            ghandle[c] = (
                pltpu.async_copy(yg_hbm.at[sl0], abufs[p], gsems[p]),
                pltpu.async_copy(yg_hbm.at[sl1], bbufs[p], gsems[p]),
            )

        start_gathers(0)
        for c in range(_CNCH):
            p = c % 2
            if c + 1 < _CNCH:
                if c + 1 >= 2:
                    shandle[c - 1].wait()
                start_gathers(c + 1)
            h0, h1 = ghandle[c]
            h0.wait()
            h1.wait()

            a, b = abufs[p], bbufs[p]

            def add_row(r, carry):
                def add_grp(q, carry2):
                    def add_col(u, carry3):
                        sl = pl.ds((q * 16 + u) * 16, 16)
                        a[r, sl] = a[r, sl] + b[r, sl]
                        return carry3
                    return lax.fori_loop(0, 16, add_col, carry2,
                                         unroll=True)
                return lax.fori_loop(0, D // 256, add_grp, carry)

            lax.fori_loop(0, _CCH, add_row, 0)
            shandle[c] = pltpu.async_copy(
                a, out_hbm.at[pl.ds(base + c * _CCH, _CCH)], ssems[p])
        for c in range(_CNCH - 2, _CNCH):
            shandle[c].wait()

    return gc_k


def _sc_gather_combine(yg, j0, j1):
    return _build_sc_gather_combine()(yg, j0, j1)


_LROWS = T // NW


@functools.cache
def _build_sc_logits_combine():
    mesh = plsc.VectorSubcoreMesh(core_axis_name="c", subcore_axis_name="s")

    @functools.partial(
        pl.kernel,
        mesh=mesh,
        out_type=jax.ShapeDtypeStruct((T, EP), jnp.float32),
        scratch_types=[
            pltpu.VMEM((_LROWS,), jnp.int32),
            pltpu.VMEM((_LROWS,), jnp.int32),
            pltpu.VMEM((_LROWS, EP), jnp.float32),
            pltpu.VMEM((_LROWS, EP), jnp.float32),
            pltpu.SemaphoreType.DMA,
        ],
    )
    def lc_k(lg_hbm, i0_hbm, i1_hbm, out_hbm, i0_v, i1_v, b0, b1, sem):
        wid = lax.axis_index("s") * NC + lax.axis_index("c")
        base = wid * _LROWS
        pltpu.async_copy(i0_hbm.at[pl.ds(base, _LROWS)], i0_v, sem).wait()
        pltpu.async_copy(i1_hbm.at[pl.ds(base, _LROWS)], i1_v, sem).wait()
        h0 = pltpu.async_copy(lg_hbm.at[i0_v], b0, sem)
        h1 = pltpu.async_copy(lg_hbm.at[i1_v], b1, sem)
        h0.wait()
        h1.wait()

        def add_row(r, carry):
            sl = pl.ds(0, 16)
            b0[r, sl] = b0[r, sl] + b1[r, sl]
            return carry

        lax.fori_loop(0, _LROWS, add_row, 0)
        pltpu.sync_copy(b0, out_hbm.at[pl.ds(base, _LROWS)])

    return lc_k


def _sc_logits_combine(lg, i0, i1):
    return _build_sc_logits_combine()(lg, i0, i1)


def _moe_body_lg(te_ref, xg_ref, win_ref, wout_ref, g_ref, wg2_ref,
                 yg_ref, lg_ref, acc_ref):
    j = pl.program_id(1)
    h = jax.nn.gelu(jnp.dot(xg_ref[...], win_ref[0],
                            preferred_element_type=jnp.float32))
    prod = jnp.dot(h, wout_ref[0], preferred_element_type=jnp.float32)

    @pl.when(j == 0)
    def _():
        acc_ref[...] = prod

    @pl.when(j > 0)
    def _():
        acc_ref[...] += prod

    @pl.when(j == NFC - 1)
    def _():
        yg = acc_ref[...] * g_ref[...]
        yg_ref[...] = yg
        lg_ref[...] = jnp.dot(yg, wg2_ref[...],
                              preferred_element_type=jnp.float32)


def _moe_body(te_ref, xg_ref, win_ref, wout_ref, g_ref, yg_ref, acc_ref):
    j = pl.program_id(1)
    h = jax.nn.gelu(jnp.dot(xg_ref[...], win_ref[0],
                            preferred_element_type=jnp.float32))
    prod = jnp.dot(h, wout_ref[0], preferred_element_type=jnp.float32)

    @pl.when(j == 0)
    def _():
        acc_ref[...] = prod

    @pl.when(j > 0)
    def _():
        acc_ref[...] += prod

    @pl.when(j == NFC - 1)
    def _():
        yg_ref[...] = acc_ref[...] * g_ref[...]


def _moe_lg(xg, W_in, W_out, gates2d, tile_expert, Wg2p):
    grid_spec = pltpu.PrefetchScalarGridSpec(
        num_scalar_prefetch=1,
        grid=(NT, NFC),
        in_specs=[
            pl.BlockSpec((TM, D), lambda i, j, te: (i, 0)),
            pl.BlockSpec((1, D, FCH), lambda i, j, te: (te[i], 0, j)),
            pl.BlockSpec((1, FCH, D), lambda i, j, te: (te[i], j, 0)),
            pl.BlockSpec((TM, 1), lambda i, j, te: (i, 0)),
            pl.BlockSpec((D, EP), lambda i, j, te: (0, 0)),
        ],
        out_specs=[
            pl.BlockSpec((TM, D), lambda i, j, te: (i, 0)),
            pl.BlockSpec((TM, EP), lambda i, j, te: (i, 0)),
        ],
        scratch_shapes=[pltpu.VMEM((TM, D), jnp.float32)],
    )
    return pl.pallas_call(
        _moe_body_lg,
        grid_spec=grid_spec,
        out_shape=[
            jax.ShapeDtypeStruct((P, D), jnp.float32),
            jax.ShapeDtypeStruct((P, EP), jnp.float32),
        ],
    )(tile_expert, xg, W_in, W_out, gates2d, Wg2p)


def _moe(xg, W_in, W_out, gates2d, tile_expert):
    grid_spec = pltpu.PrefetchScalarGridSpec(
        num_scalar_prefetch=1,
        grid=(NT, NFC),
        in_specs=[
            pl.BlockSpec((TM, D), lambda i, j, te: (i, 0)),
            pl.BlockSpec((1, D, FCH), lambda i, j, te: (te[i], 0, j)),
            pl.BlockSpec((1, FCH, D), lambda i, j, te: (te[i], j, 0)),
            pl.BlockSpec((TM, 1), lambda i, j, te: (i, 0)),
        ],
        out_specs=pl.BlockSpec((TM, D), lambda i, j, te: (i, 0)),
        scratch_shapes=[pltpu.VMEM((TM, D), jnp.float32)],
    )
    return pl.pallas_call(
        _moe_body,
        grid_spec=grid_spec,
        out_shape=jax.ShapeDtypeStruct((P, D), jnp.float32),
    )(tile_expert, xg, W_in, W_out, gates2d)


def _final_body(y_ref, yg_ref, bf_ref, sent_ref, out_ref, acc_ref):
    i = pl.program_id(0)

    @pl.when(i == 0)
    def _():
        acc_ref[...] = jnp.zeros_like(acc_ref)

    rows = yg_ref[...]
    bf = bf_ref[...]
    part1 = jnp.sum(rows * bf, axis=0, keepdims=True)
    part_all = jnp.sum(rows, axis=0, keepdims=True)
    part0 = part_all - part1
    rowi = lax.broadcasted_iota(jnp.int32, (8, D), 0)
    acc_ref[...] += jnp.where(rowi == 0, part0, 0.0)
    acc_ref[...] += jnp.where(rowi == 1, part1, 0.0)

    @pl.when(i == NT - 1)
    def _():
        sent = (acc_ref[...] + sent_ref[...]) / jnp.float32(S)
        mx = jnp.max(sent, axis=1, keepdims=True)
        z = sent - mx
        lse = jnp.log(jnp.sum(jnp.exp(z), axis=1, keepdims=True))
        logp = z - lse
        coli = lax.broadcasted_iota(jnp.int32, (8, D), 1)
        rowj = lax.broadcasted_iota(jnp.int32, (8, D), 0)
        sel = (((rowj == 0) & (coli == y_ref[0]))
               | ((rowj == 1) & (coli == y_ref[1])))
        loss = -jnp.sum(jnp.where(sel, logp, 0.0)) / jnp.float32(B)
        out_ref[...] = jnp.full((8, 128), loss, jnp.float32)


def _final(yg2, bflag, sent_lin, y):
    grid_spec = pltpu.PrefetchScalarGridSpec(
        num_scalar_prefetch=1,
        grid=(NT,),
        in_specs=[
            pl.BlockSpec((TM, D), lambda i, y_ref: (i, 0)),
            pl.BlockSpec((TM, 1), lambda i, y_ref: (i, 0)),
            pl.BlockSpec((8, D), lambda i, y_ref: (0, 0)),
        ],
        out_specs=pl.BlockSpec((8, 128), lambda i, y_ref: (0, 0)),
        scratch_shapes=[pltpu.VMEM((8, D), jnp.float32)],
    )
    return pl.pallas_call(
        _final_body,
        grid_spec=grid_spec,
        out_shape=jax.ShapeDtypeStruct((8, 128), jnp.float32),
    )(y, yg2, bflag, sent_lin)


def kernel(x, y, W_lin, b_lin, Wg1, W1_in, W1_out, Wg2, W2_in, W2_out):
    x2 = x.reshape(T, D)
    flat, logits1, sent_lin = _linear(x2, W_lin, b_lin, Wg1)

    row1, g1, te1, i10, i11 = _route(logits1)
    xg1 = _sc_gather(flat, row1)
    Wg2p = jnp.pad(Wg2, ((0, 0), (0, EP - E)))
    yg1, lg1 = _moe_lg(xg1, W1_in, W1_out, g1, te1, Wg2p)

    logits2 = _sc_logits_combine(lg1, i10, i11)[:, :E]
    row2, g2, te2, _, _ = _route(logits2)
    j0 = i10[row2]
    j1 = i11[row2]
    bflag = (row2 >= S).astype(jnp.float32).reshape(P, 1)

    xg2 = _sc_gather_combine(yg1, j0, j1)
    yg2 = _moe(xg2, W2_in, W2_out, g2, te2)

    loss = _final(yg2, bflag, sent_lin, y.astype(jnp.int32))
    return loss[0, 0]


# speedup vs baseline: 1.0868x; 1.0868x over previous
import functools

import jax
import jax.numpy as jnp
from jax import lax
from jax.experimental import pallas as pl
from jax.experimental.pallas import tpu as pltpu
from jax.experimental.pallas import tpu_sc as plsc

B = 2
S = 2048
T = B * S
D = 1024
F = 2048
E = 8
K = 2
A = T * K
EP = 128

TM = 256
P = A + E * TM
NT = P // TM
FCH = 512
NFC = F // FCH

NC = 2
NS = 16
NW = NC * NS


def _linear_body(x_ref, wl_ref, b_ref, wg_ref, flat_ref, log_ref, sent_ref,
                 acc_ref):
    i = pl.program_id(0)

    @pl.when(i == 0)
    def _():
        acc_ref[...] = jnp.zeros_like(acc_ref)

    acc = jnp.dot(x_ref[...], wl_ref[...],
                  preferred_element_type=jnp.float32) + b_ref[...]
    flat_ref[...] = acc
    log_ref[...] = jnp.dot(acc, wg_ref[...],
                           preferred_element_type=jnp.float32)

    part = jnp.sum(acc, axis=0, keepdims=True)
    b = i // (S // TM)
    rowi = lax.broadcasted_iota(jnp.int32, (8, D), 0)
    acc_ref[...] += jnp.where(rowi == b, part, 0.0)

    @pl.when(i == T // TM - 1)
    def _():
        sent_ref[...] = acc_ref[...]


def _linear(x2, W_lin, b_lin, Wg1):
    return pl.pallas_call(
        _linear_body,
        grid=(T // TM,),
        in_specs=[
            pl.BlockSpec((TM, D), lambda i: (i, 0)),
            pl.BlockSpec((D, D), lambda i: (0, 0)),
            pl.BlockSpec((1, D), lambda i: (0, 0)),
            pl.BlockSpec((D, E), lambda i: (0, 0)),
        ],
        out_specs=[
            pl.BlockSpec((TM, D), lambda i: (i, 0)),
            pl.BlockSpec((TM, E), lambda i: (i, 0)),
            pl.BlockSpec((8, D), lambda i: (0, 0)),
        ],
        out_shape=[
            jax.ShapeDtypeStruct((T, D), jnp.float32),
            jax.ShapeDtypeStruct((T, E), jnp.float32),
            jax.ShapeDtypeStruct((8, D), jnp.float32),
        ],
        scratch_shapes=[pltpu.VMEM((8, D), jnp.float32)],
    )(x2, W_lin, b_lin.reshape(1, D), Wg1)


def _route(logits):
    probs = jax.nn.softmax(logits, axis=-1)
    topv, topi = lax.top_k(probs, K)
    gates = topv / jnp.sum(topv, axis=-1, keepdims=True)

    e = topi.reshape(-1).astype(jnp.int32)
    g = gates.reshape(-1)
    oh = (e[:, None] == jnp.arange(E, dtype=jnp.int32)).astype(jnp.int32)
    cum = jnp.cumsum(oh, axis=0)
    rank = cum[jnp.arange(A), e] - 1
    counts = cum[-1]
    padded = ((counts + TM - 1) // TM) * TM
    ends = jnp.cumsum(padded)
    gstart = ends - padded
    dest = (gstart[e] + rank).astype(jnp.int32)

    tok = jnp.arange(A, dtype=jnp.int32) // K
    row_index = jnp.zeros((P,), jnp.int32).at[dest].set(tok)
    gate_s = jnp.zeros((P,), jnp.float32).at[dest].set(g)
    tile_expert = jnp.searchsorted(
        ends, jnp.arange(NT, dtype=jnp.int32) * TM, side='right')
    tile_expert = jnp.minimum(tile_expert, E - 1).astype(jnp.int32)
    inv = dest.reshape(T, K)
    return row_index, gate_s.reshape(P, 1), tile_expert, inv[:, 0], inv[:, 1]


_GCH = 40
_GROWS = P // NW
_GNCH = _GROWS // _GCH


@functools.cache
def _build_sc_gather():
    mesh = plsc.VectorSubcoreMesh(core_axis_name="c", subcore_axis_name="s")

    @functools.partial(
        pl.kernel,
        mesh=mesh,
        out_type=jax.ShapeDtypeStruct((P, D), jnp.float32),
        scratch_types=[
            [pltpu.VMEM((_GCH,), jnp.int32) for _ in range(_GNCH)],
            [pltpu.VMEM((_GCH, D), jnp.float32) for _ in range(3)],
            pltpu.SemaphoreType.DMA,
            [pltpu.SemaphoreType.DMA for _ in range(3)],
            [pltpu.SemaphoreType.DMA for _ in range(3)],
        ],
    )
    def gather_k(src_hbm, idx_hbm, out_hbm, idx_bufs, bufs, isem, gsems,
                 ssems):
        wid = lax.axis_index("s") * NC + lax.axis_index("c")
        base = wid * _GROWS
        ih = [pltpu.async_copy(
                  idx_hbm.at[pl.ds(base + c * _GCH, _GCH)], idx_bufs[c],
                  isem)
              for c in range(_GNCH)]
        for h in ih:
            h.wait()

        ghandle = {}
        shandle = {}

        def start_gather(c):
            ghandle[c] = pltpu.async_copy(
                src_hbm.at[idx_bufs[c]], bufs[c % 3], gsems[c % 3])

        start_gather(0)
        for c in range(_GNCH):
            if c + 1 < _GNCH:
                if c + 1 >= 3:
                    shandle[c + 1 - 3].wait()
                start_gather(c + 1)
            ghandle[c].wait()
            shandle[c] = pltpu.async_copy(
                bufs[c % 3], out_hbm.at[pl.ds(base + c * _GCH, _GCH)],
                ssems[c % 3])
        for c in range(_GNCH - 3, _GNCH):
            shandle[c].wait()

    return gather_k


def _sc_gather(src, idx):
    return _build_sc_gather()(src, idx)


_CCH = 16
_CNCH = _GROWS // _CCH


@functools.cache
def _build_sc_gather_combine():
    mesh = plsc.VectorSubcoreMesh(core_axis_name="c", subcore_axis_name="s")

    @functools.partial(
        pl.kernel,
        mesh=mesh,
        out_type=jax.ShapeDtypeStruct((P, D), jnp.float32),
        scratch_types=[
            [pltpu.VMEM((2 * _CCH,), jnp.int32) for _ in range(_CNCH)],
            [pltpu.VMEM((2 * _CCH, D), jnp.float32) for _ in range(3)],
            pltpu.SemaphoreType.DMA,
            [pltpu.SemaphoreType.DMA for _ in range(3)],
            [pltpu.SemaphoreType.DMA for _ in range(3)],
        ],
    )
    def gc_k(yg_hbm, jj_hbm, out_hbm, idx_bufs, bufs, isem, gsems, ssems):
        wid = lax.axis_index("s") * NC + lax.axis_index("c")
        base = wid * _GROWS
        ih = [pltpu.async_copy(
                  jj_hbm.at[pl.ds(2 * (base + c * _CCH), 2 * _CCH)],
                  idx_bufs[c], isem)
              for c in range(_CNCH)]
        for h in ih:
            h.wait()

        ghandle = {}
        shandle = {}

        def start_gather(c):
            ghandle[c] = pltpu.async_copy(
                yg_hbm.at[idx_bufs[c]], bufs[c % 3], gsems[c % 3])

        start_gather(0)
        for c in range(_CNCH):
            if c + 1 < _CNCH:
                if c + 1 >= 3:
                    shandle[c + 1 - 3].wait()
                start_gather(c + 1)
            ghandle[c].wait()
            buf = bufs[c % 3]

            def add_row(r, carry):
                def add_grp(q, carry2):
                    def add_col(u, carry3):
                        sl = pl.ds((q * 16 + u) * 16, 16)
                        buf[r, sl] = buf[r, sl] + buf[r + _CCH, sl]
                        return carry3
                    return lax.fori_loop(0, 16, add_col, carry2,
                                         unroll=True)
                return lax.fori_loop(0, D // 256, add_grp, carry)

            lax.fori_loop(0, _CCH, add_row, 0)
            shandle[c] = pltpu.async_copy(
                buf.at[pl.ds(0, _CCH)],
                out_hbm.at[pl.ds(base + c * _CCH, _CCH)], ssems[c % 3])
        for c in range(_CNCH - 3, _CNCH):
            shandle[c].wait()

    return gc_k


def _sc_gather_combine(yg, jj):
    return _build_sc_gather_combine()(yg, jj)


_LROWS = T // NW


@functools.cache
def _build_sc_logits_combine():
    mesh = plsc.VectorSubcoreMesh(core_axis_name="c", subcore_axis_name="s")

    @functools.partial(
        pl.kernel,
        mesh=mesh,
        out_type=jax.ShapeDtypeStruct((T, EP), jnp.float32),
        scratch_types=[
            pltpu.VMEM((_LROWS,), jnp.int32),
            pltpu.VMEM((_LROWS,), jnp.int32),
            pltpu.VMEM((_LROWS, EP), jnp.float32),
            pltpu.VMEM((_LROWS, EP), jnp.float32),
            pltpu.SemaphoreType.DMA,
        ],
    )
    def lc_k(lg_hbm, i0_hbm, i1_hbm, out_hbm, i0_v, i1_v, b0, b1, sem):
        wid = lax.axis_index("s") * NC + lax.axis_index("c")
        base = wid * _LROWS
        pltpu.async_copy(i0_hbm.at[pl.ds(base, _LROWS)], i0_v, sem).wait()
        pltpu.async_copy(i1_hbm.at[pl.ds(base, _LROWS)], i1_v, sem).wait()
        h0 = pltpu.async_copy(lg_hbm.at[i0_v], b0, sem)
        h1 = pltpu.async_copy(lg_hbm.at[i1_v], b1, sem)
        h0.wait()
        h1.wait()

        def add_row(r, carry):
            sl = pl.ds(0, 16)
            b0[r, sl] = b0[r, sl] + b1[r, sl]
            return carry

        lax.fori_loop(0, _LROWS, add_row, 0)
        pltpu.sync_copy(b0, out_hbm.at[pl.ds(base, _LROWS)])

    return lc_k


def _sc_logits_combine(lg, i0, i1):
    return _build_sc_logits_combine()(lg, i0, i1)


def _moe_body_lg(te_ref, xg_ref, win_ref, wout_ref, g_ref, wg2_ref,
                 yg_ref, lg_ref, acc_ref):
    j = pl.program_id(1)
    h = jax.nn.gelu(jnp.dot(xg_ref[...], win_ref[0],
                            preferred_element_type=jnp.float32))
    prod = jnp.dot(h, wout_ref[0], preferred_element_type=jnp.float32)

    @pl.when(j == 0)
    def _():
        acc_ref[...] = prod

    @pl.when(j > 0)
    def _():
        acc_ref[...] += prod

    @pl.when(j == NFC - 1)
    def _():
        yg = acc_ref[...] * g_ref[...]
        yg_ref[...] = yg
        lg_ref[...] = jnp.dot(yg, wg2_ref[...],
                              preferred_element_type=jnp.float32)


def _moe_body(te_ref, xg_ref, win_ref, wout_ref, g_ref, yg_ref, acc_ref):
    j = pl.program_id(1)
    h = jax.nn.gelu(jnp.dot(xg_ref[...], win_ref[0],
                            preferred_element_type=jnp.float32))
    prod = jnp.dot(h, wout_ref[0], preferred_element_type=jnp.float32)

    @pl.when(j == 0)
    def _():
        acc_ref[...] = prod

    @pl.when(j > 0)
    def _():
        acc_ref[...] += prod

    @pl.when(j == NFC - 1)
    def _():
        yg_ref[...] = acc_ref[...] * g_ref[...]


def _moe_lg(xg, W_in, W_out, gates2d, tile_expert, Wg2p):
    grid_spec = pltpu.PrefetchScalarGridSpec(
        num_scalar_prefetch=1,
        grid=(NT, NFC),
        in_specs=[
            pl.BlockSpec((TM, D), lambda i, j, te: (i, 0)),
            pl.BlockSpec((1, D, FCH), lambda i, j, te: (te[i], 0, j)),
            pl.BlockSpec((1, FCH, D), lambda i, j, te: (te[i], j, 0)),
            pl.BlockSpec((TM, 1), lambda i, j, te: (i, 0)),
            pl.BlockSpec((D, EP), lambda i, j, te: (0, 0)),
        ],
        out_specs=[
            pl.BlockSpec((TM, D), lambda i, j, te: (i, 0)),
            pl.BlockSpec((TM, EP), lambda i, j, te: (i, 0)),
        ],
        scratch_shapes=[pltpu.VMEM((TM, D), jnp.float32)],
    )
    return pl.pallas_call(
        _moe_body_lg,
        grid_spec=grid_spec,
        out_shape=[
            jax.ShapeDtypeStruct((P, D), jnp.float32),
            jax.ShapeDtypeStruct((P, EP), jnp.float32),
        ],
    )(tile_expert, xg, W_in, W_out, gates2d, Wg2p)


def _moe(xg, W_in, W_out, gates2d, tile_expert):
    grid_spec = pltpu.PrefetchScalarGridSpec(
        num_scalar_prefetch=1,
        grid=(NT, NFC),
        in_specs=[
            pl.BlockSpec((TM, D), lambda i, j, te: (i, 0)),
            pl.BlockSpec((1, D, FCH), lambda i, j, te: (te[i], 0, j)),
            pl.BlockSpec((1, FCH, D), lambda i, j, te: (te[i], j, 0)),
            pl.BlockSpec((TM, 1), lambda i, j, te: (i, 0)),
        ],
        out_specs=pl.BlockSpec((TM, D), lambda i, j, te: (i, 0)),
        scratch_shapes=[pltpu.VMEM((TM, D), jnp.float32)],
    )
    return pl.pallas_call(
        _moe_body,
        grid_spec=grid_spec,
        out_shape=jax.ShapeDtypeStruct((P, D), jnp.float32),
    )(tile_expert, xg, W_in, W_out, gates2d)


def _final_body(y_ref, yg_ref, bf_ref, sent_ref, out_ref, acc_ref):
    i = pl.program_id(0)

    @pl.when(i == 0)
    def _():
        acc_ref[...] = jnp.zeros_like(acc_ref)

    rows = yg_ref[...]
    bf = bf_ref[...]
    part1 = jnp.sum(rows * bf, axis=0, keepdims=True)
    part_all = jnp.sum(rows, axis=0, keepdims=True)
    part0 = part_all - part1
    rowi = lax.broadcasted_iota(jnp.int32, (8, D), 0)
    acc_ref[...] += jnp.where(rowi == 0, part0, 0.0)
    acc_ref[...] += jnp.where(rowi == 1, part1, 0.0)

    @pl.when(i == NT - 1)
    def _():
        sent = (acc_ref[...] + sent_ref[...]) / jnp.float32(S)
        mx = jnp.max(sent, axis=1, keepdims=True)
        z = sent - mx
        lse = jnp.log(jnp.sum(jnp.exp(z), axis=1, keepdims=True))
        logp = z - lse
        coli = lax.broadcasted_iota(jnp.int32, (8, D), 1)
        rowj = lax.broadcasted_iota(jnp.int32, (8, D), 0)
        sel = (((rowj == 0) & (coli == y_ref[0]))
               | ((rowj == 1) & (coli == y_ref[1])))
        loss = -jnp.sum(jnp.where(sel, logp, 0.0)) / jnp.float32(B)
        out_ref[...] = jnp.full((8, 128), loss, jnp.float32)


def _final(yg2, bflag, sent_lin, y):
    grid_spec = pltpu.PrefetchScalarGridSpec(
        num_scalar_prefetch=1,
        grid=(NT,),
        in_specs=[
            pl.BlockSpec((TM, D), lambda i, y_ref: (i, 0)),
            pl.BlockSpec((TM, 1), lambda i, y_ref: (i, 0)),
            pl.BlockSpec((8, D), lambda i, y_ref: (0, 0)),
        ],
        out_specs=pl.BlockSpec((8, 128), lambda i, y_ref: (0, 0)),
        scratch_shapes=[pltpu.VMEM((8, D), jnp.float32)],
    )
    return pl.pallas_call(
        _final_body,
        grid_spec=grid_spec,
        out_shape=jax.ShapeDtypeStruct((8, 128), jnp.float32),
    )(y, yg2, bflag, sent_lin)


def kernel(x, y, W_lin, b_lin, Wg1, W1_in, W1_out, Wg2, W2_in, W2_out):
    x2 = x.reshape(T, D)
    flat, logits1, sent_lin = _linear(x2, W_lin, b_lin, Wg1)

    row1, g1, te1, i10, i11 = _route(logits1)
    xg1 = _sc_gather(flat, row1)
    Wg2p = jnp.pad(Wg2, ((0, 0), (0, EP - E)))
    yg1, lg1 = _moe_lg(xg1, W1_in, W1_out, g1, te1, Wg2p)

    logits2 = _sc_logits_combine(lg1, i10, i11)[:, :E]
    row2, g2, te2, _, _ = _route(logits2)
    j0 = i10[row2]
    j1 = i11[row2]
    jj = jnp.stack([j0.reshape(-1, _CCH), j1.reshape(-1, _CCH)],
                   axis=1).reshape(-1)
    bflag = (row2 >= S).astype(jnp.float32).reshape(P, 1)

    xg2 = _sc_gather_combine(yg1, jj)
    yg2 = _moe(xg2, W2_in, W2_out, g2, te2)

    loss = _final(yg2, bflag, sent_lin, y.astype(jnp.int32))
    return loss[0, 0]


# speedup vs baseline: 1.4195x; 1.3060x over previous
import functools

import jax
import jax.numpy as jnp
from jax import lax
from jax.experimental import pallas as pl
from jax.experimental.pallas import tpu as pltpu
from jax.experimental.pallas import tpu_sc as plsc

B = 2
S = 2048
T = B * S
D = 1024
F = 2048
E = 8
K = 2
A = T * K
EP = 128

TM = 256
P = A + E * TM
NT = P // TM
FCH = 512
NFC = F // FCH

NC = 2
NS = 16
NW = NC * NS


def _linear_body(x_ref, wl_ref, b_ref, wg_ref, flat_ref, log_ref, sent_ref,
                 acc_ref):
    i = pl.program_id(0)

    @pl.when(i == 0)
    def _():
        acc_ref[...] = jnp.zeros_like(acc_ref)

    acc = jnp.dot(x_ref[...], wl_ref[...],
                  preferred_element_type=jnp.float32) + b_ref[...]
    flat_ref[...] = acc
    log_ref[...] = jnp.dot(acc, wg_ref[...],
                           preferred_element_type=jnp.float32)

    part = jnp.sum(acc, axis=0, keepdims=True)
    b = i // (S // TM)
    rowi = lax.broadcasted_iota(jnp.int32, (8, D), 0)
    acc_ref[...] += jnp.where(rowi == b, part, 0.0)

    @pl.when(i == T // TM - 1)
    def _():
        sent_ref[...] = acc_ref[...]


def _linear(x2, W_lin, b_lin, Wg1):
    return pl.pallas_call(
        _linear_body,
        grid=(T // TM,),
        in_specs=[
            pl.BlockSpec((TM, D), lambda i: (i, 0)),
            pl.BlockSpec((D, D), lambda i: (0, 0)),
            pl.BlockSpec((1, D), lambda i: (0, 0)),
            pl.BlockSpec((D, E), lambda i: (0, 0)),
        ],
        out_specs=[
            pl.BlockSpec((TM, D), lambda i: (i, 0)),
            pl.BlockSpec((TM, E), lambda i: (i, 0)),
            pl.BlockSpec((8, D), lambda i: (0, 0)),
        ],
        out_shape=[
            jax.ShapeDtypeStruct((T, D), jnp.float32),
            jax.ShapeDtypeStruct((T, E), jnp.float32),
            jax.ShapeDtypeStruct((8, D), jnp.float32),
        ],
        scratch_shapes=[pltpu.VMEM((8, D), jnp.float32)],
    )(x2, W_lin, b_lin.reshape(1, D), Wg1)


def _route(logits):
    probs = jax.nn.softmax(logits, axis=-1)
    topv, topi = lax.top_k(probs, K)
    gates = topv / jnp.sum(topv, axis=-1, keepdims=True)

    e = topi.reshape(-1).astype(jnp.int32)
    g = gates.reshape(-1)
    oh = (e[:, None] == jnp.arange(E, dtype=jnp.int32)).astype(jnp.int32)
    cum = jnp.cumsum(oh, axis=0)
    rank = cum[jnp.arange(A), e] - 1
    counts = cum[-1]
    padded = ((counts + TM - 1) // TM) * TM
    ends = jnp.cumsum(padded)
    gstart = ends - padded
    dest = (gstart[e] + rank).astype(jnp.int32)

    tok = jnp.arange(A, dtype=jnp.int32) // K
    row_index = jnp.zeros((P,), jnp.int32).at[dest].set(tok)
    gate_s = jnp.zeros((P,), jnp.float32).at[dest].set(g)
    tile_expert = jnp.searchsorted(
        ends, jnp.arange(NT, dtype=jnp.int32) * TM, side='right')
    tile_expert = jnp.minimum(tile_expert, E - 1).astype(jnp.int32)
    inv = dest.reshape(T, K)
    return row_index, gate_s.reshape(P, 1), tile_expert, inv[:, 0], inv[:, 1]


_GCH = 40
_GROWS = P // NW
_GNCH = _GROWS // _GCH


@functools.cache
def _build_sc_gather():
    mesh = plsc.VectorSubcoreMesh(core_axis_name="c", subcore_axis_name="s")

    @functools.partial(
        pl.kernel,
        mesh=mesh,
        out_type=jax.ShapeDtypeStruct((P, D), jnp.float32),
        scratch_types=[
            [pltpu.VMEM((_GCH,), jnp.int32) for _ in range(_GNCH)],
            [pltpu.VMEM((_GCH, D), jnp.float32) for _ in range(3)],
            pltpu.SemaphoreType.DMA,
            [pltpu.SemaphoreType.DMA for _ in range(3)],
            [pltpu.SemaphoreType.DMA for _ in range(3)],
        ],
    )
    def gather_k(src_hbm, idx_hbm, out_hbm, idx_bufs, bufs, isem, gsems,
                 ssems):
        wid = lax.axis_index("s") * NC + lax.axis_index("c")
        base = wid * _GROWS
        ih = [pltpu.async_copy(
                  idx_hbm.at[pl.ds(base + c * _GCH, _GCH)], idx_bufs[c],
                  isem)
              for c in range(_GNCH)]
        for h in ih:
            h.wait()

        ghandle = {}
        shandle = {}

        def start_gather(c):
            ghandle[c] = pltpu.async_copy(
                src_hbm.at[idx_bufs[c]], bufs[c % 3], gsems[c % 3])

        start_gather(0)
        for c in range(_GNCH):
            if c + 1 < _GNCH:
                if c + 1 >= 3:
                    shandle[c + 1 - 3].wait()
                start_gather(c + 1)
            ghandle[c].wait()
            shandle[c] = pltpu.async_copy(
                bufs[c % 3], out_hbm.at[pl.ds(base + c * _GCH, _GCH)],
                ssems[c % 3])
        for c in range(_GNCH - 3, _GNCH):
            shandle[c].wait()

    return gather_k


def _sc_gather(src, idx):
    return _build_sc_gather()(src, idx)


_CCH = 16
_CNCH = _GROWS // _CCH


@functools.cache
def _build_sc_gather_combine():
    mesh = plsc.VectorSubcoreMesh(core_axis_name="c", subcore_axis_name="s")

    @functools.partial(
        pl.kernel,
        mesh=mesh,
        out_type=jax.ShapeDtypeStruct((P, D), jnp.float32),
        scratch_types=[
            [pltpu.VMEM((2 * _CCH,), jnp.int32) for _ in range(_CNCH)],
            [pltpu.VMEM((2 * _CCH, D), jnp.float32) for _ in range(3)],
            pltpu.SemaphoreType.DMA,
            [pltpu.SemaphoreType.DMA for _ in range(3)],
            [pltpu.SemaphoreType.DMA for _ in range(3)],
        ],
    )
    def gc_k(yg_hbm, jj_hbm, out_hbm, idx_bufs, bufs, isem, gsems, ssems):
        wid = lax.axis_index("s") * NC + lax.axis_index("c")
        base = wid * _GROWS
        ih = [pltpu.async_copy(
                  jj_hbm.at[pl.ds(2 * (base + c * _CCH), 2 * _CCH)],
                  idx_bufs[c], isem)
              for c in range(_CNCH)]
        for h in ih:
            h.wait()

        ghandle = {}
        shandle = {}

        def start_gather(c):
            ghandle[c] = pltpu.async_copy(
                yg_hbm.at[idx_bufs[c]], bufs[c % 3], gsems[c % 3])

        start_gather(0)
        for c in range(_CNCH):
            if c + 1 < _CNCH:
                if c + 1 >= 3:
                    shandle[c + 1 - 3].wait()
                start_gather(c + 1)
            ghandle[c].wait()
            buf = bufs[c % 3]

            def add_row(r, carry):
                def add_grp(q, carry2):
                    def add_col(u, carry3):
                        sl = pl.ds((q * 16 + u) * 16, 16)
                        buf[r, sl] = buf[r, sl] + buf[r + _CCH, sl]
                        return carry3
                    return lax.fori_loop(0, 16, add_col, carry2,
                                         unroll=True)
                return lax.fori_loop(0, D // 256, add_grp, carry)

            lax.fori_loop(0, _CCH, add_row, 0)
            shandle[c] = pltpu.async_copy(
                buf.at[pl.ds(0, _CCH)],
                out_hbm.at[pl.ds(base + c * _CCH, _CCH)], ssems[c % 3])
        for c in range(_CNCH - 3, _CNCH):
            shandle[c].wait()

    return gc_k


def _sc_gather_combine(yg, jj):
    return _build_sc_gather_combine()(yg, jj)


_LROWS = T // NW


@functools.cache
def _build_sc_logits_combine():
    mesh = plsc.VectorSubcoreMesh(core_axis_name="c", subcore_axis_name="s")

    @functools.partial(
        pl.kernel,
        mesh=mesh,
        out_type=jax.ShapeDtypeStruct((T, EP), jnp.float32),
        scratch_types=[
            pltpu.VMEM((_LROWS,), jnp.int32),
            pltpu.VMEM((_LROWS,), jnp.int32),
            pltpu.VMEM((_LROWS, EP), jnp.float32),
            pltpu.VMEM((_LROWS, EP), jnp.float32),
            pltpu.SemaphoreType.DMA,
        ],
    )
    def lc_k(lg_hbm, i0_hbm, i1_hbm, out_hbm, i0_v, i1_v, b0, b1, sem):
        wid = lax.axis_index("s") * NC + lax.axis_index("c")
        base = wid * _LROWS
        pltpu.async_copy(i0_hbm.at[pl.ds(base, _LROWS)], i0_v, sem).wait()
        pltpu.async_copy(i1_hbm.at[pl.ds(base, _LROWS)], i1_v, sem).wait()
        h0 = pltpu.async_copy(lg_hbm.at[i0_v], b0, sem)
        h1 = pltpu.async_copy(lg_hbm.at[i1_v], b1, sem)
        h0.wait()
        h1.wait()

        def add_row(r, carry):
            sl = pl.ds(0, 16)
            b0[r, sl] = b0[r, sl] + b1[r, sl]
            return carry

        lax.fori_loop(0, _LROWS, add_row, 0)
        pltpu.sync_copy(b0, out_hbm.at[pl.ds(base, _LROWS)])

    return lc_k


def _sc_logits_combine(lg, i0, i1):
    return _build_sc_logits_combine()(lg, i0, i1)


def _moe_body_lg(te_ref, xg_ref, win_ref, wout_ref, g_ref, wg2_ref,
                 yg_ref, lg_ref):
    h = jax.nn.gelu(jnp.dot(xg_ref[...], win_ref[0],
                            preferred_element_type=jnp.float32))
    yg = jnp.dot(h, wout_ref[0], preferred_element_type=jnp.float32)
    yg = yg * g_ref[...]
    yg_ref[...] = yg
    lg_ref[...] = jnp.dot(yg, wg2_ref[...], preferred_element_type=jnp.float32)


def _moe_body(te_ref, xg_ref, win_ref, wout_ref, g_ref, yg_ref):
    h = jax.nn.gelu(jnp.dot(xg_ref[...], win_ref[0],
                            preferred_element_type=jnp.float32))
    yg = jnp.dot(h, wout_ref[0], preferred_element_type=jnp.float32)
    yg_ref[...] = yg * g_ref[...]


def _moe_lg(xg, W_in, W_out, gates2d, tile_expert, Wg2p):
    grid_spec = pltpu.PrefetchScalarGridSpec(
        num_scalar_prefetch=1,
        grid=(NT,),
        in_specs=[
            pl.BlockSpec((TM, D), lambda i, te: (i, 0)),
            pl.BlockSpec((1, D, F), lambda i, te: (te[i], 0, 0)),
            pl.BlockSpec((1, F, D), lambda i, te: (te[i], 0, 0)),
            pl.BlockSpec((TM, 1), lambda i, te: (i, 0)),
            pl.BlockSpec((D, EP), lambda i, te: (0, 0)),
        ],
        out_specs=[
            pl.BlockSpec((TM, D), lambda i, te: (i, 0)),
            pl.BlockSpec((TM, EP), lambda i, te: (i, 0)),
        ],
    )
    return pl.pallas_call(
        _moe_body_lg,
        grid_spec=grid_spec,
        out_shape=[
            jax.ShapeDtypeStruct((P, D), jnp.float32),
            jax.ShapeDtypeStruct((P, EP), jnp.float32),
        ],
    )(tile_expert, xg, W_in, W_out, gates2d, Wg2p)


def _moe(xg, W_in, W_out, gates2d, tile_expert):
    grid_spec = pltpu.PrefetchScalarGridSpec(
        num_scalar_prefetch=1,
        grid=(NT,),
        in_specs=[
            pl.BlockSpec((TM, D), lambda i, te: (i, 0)),
            pl.BlockSpec((1, D, F), lambda i, te: (te[i], 0, 0)),
            pl.BlockSpec((1, F, D), lambda i, te: (te[i], 0, 0)),
            pl.BlockSpec((TM, 1), lambda i, te: (i, 0)),
        ],
        out_specs=pl.BlockSpec((TM, D), lambda i, te: (i, 0)),
    )
    return pl.pallas_call(
        _moe_body,
        grid_spec=grid_spec,
        out_shape=jax.ShapeDtypeStruct((P, D), jnp.float32),
    )(tile_expert, xg, W_in, W_out, gates2d)


def _final_body(y_ref, yg_ref, bf_ref, sent_ref, out_ref, acc_ref):
    i = pl.program_id(0)

    @pl.when(i == 0)
    def _():
        acc_ref[...] = jnp.zeros_like(acc_ref)

    rows = yg_ref[...]
    bf = bf_ref[...]
    part1 = jnp.sum(rows * bf, axis=0, keepdims=True)
    part_all = jnp.sum(rows, axis=0, keepdims=True)
    part0 = part_all - part1
    rowi = lax.broadcasted_iota(jnp.int32, (8, D), 0)
    acc_ref[...] += jnp.where(rowi == 0, part0, 0.0)
    acc_ref[...] += jnp.where(rowi == 1, part1, 0.0)

    @pl.when(i == NT - 1)
    def _():
        sent = (acc_ref[...] + sent_ref[...]) / jnp.float32(S)
        mx = jnp.max(sent, axis=1, keepdims=True)
        z = sent - mx
        lse = jnp.log(jnp.sum(jnp.exp(z), axis=1, keepdims=True))
        logp = z - lse
        coli = lax.broadcasted_iota(jnp.int32, (8, D), 1)
        rowj = lax.broadcasted_iota(jnp.int32, (8, D), 0)
        sel = (((rowj == 0) & (coli == y_ref[0]))
               | ((rowj == 1) & (coli == y_ref[1])))
        loss = -jnp.sum(jnp.where(sel, logp, 0.0)) / jnp.float32(B)
        out_ref[...] = jnp.full((8, 128), loss, jnp.float32)


def _final(yg2, bflag, sent_lin, y):
    grid_spec = pltpu.PrefetchScalarGridSpec(
        num_scalar_prefetch=1,
        grid=(NT,),
        in_specs=[
            pl.BlockSpec((TM, D), lambda i, y_ref: (i, 0)),
            pl.BlockSpec((TM, 1), lambda i, y_ref: (i, 0)),
            pl.BlockSpec((8, D), lambda i, y_ref: (0, 0)),
        ],
        out_specs=pl.BlockSpec((8, 128), lambda i, y_ref: (0, 0)),
        scratch_shapes=[pltpu.VMEM((8, D), jnp.float32)],
    )
    return pl.pallas_call(
        _final_body,
        grid_spec=grid_spec,
        out_shape=jax.ShapeDtypeStruct((8, 128), jnp.float32),
    )(y, yg2, bflag, sent_lin)


def kernel(x, y, W_lin, b_lin, Wg1, W1_in, W1_out, Wg2, W2_in, W2_out):
    x2 = x.reshape(T, D)
    flat, logits1, sent_lin = _linear(x2, W_lin, b_lin, Wg1)

    row1, g1, te1, i10, i11 = _route(logits1)
    xg1 = _sc_gather(flat, row1)
    Wg2p = jnp.pad(Wg2, ((0, 0), (0, EP - E)))
    yg1, lg1 = _moe_lg(xg1, W1_in, W1_out, g1, te1, Wg2p)

    logits2 = _sc_logits_combine(lg1, i10, i11)[:, :E]
    row2, g2, te2, _, _ = _route(logits2)
    j0 = i10[row2]
    j1 = i11[row2]
    jj = jnp.stack([j0.reshape(-1, _CCH), j1.reshape(-1, _CCH)],
                   axis=1).reshape(-1)
    bflag = (row2 >= S).astype(jnp.float32).reshape(P, 1)

    xg2 = _sc_gather_combine(yg1, jj)
    yg2 = _moe(xg2, W2_in, W2_out, g2, te2)

    loss = _final(yg2, bflag, sent_lin, y.astype(jnp.int32))
    return loss[0, 0]
